# Initial kernel scaffold; baseline (speedup 1.0000x reference)
import jax
import jax.numpy as jnp
from jax.experimental import pallas as pl


def kernel(x, edge_index, W1, b1, g1, be1, W2, b2, g2, be2, W3, b3):
    raise NotImplementedError("write your pallas kernel here")


# speedup vs baseline: 1.6146x; 1.6146x over previous
import jax
import jax.numpy as jnp
from jax.experimental import pallas as pl


def _ls_kernel(h_ref, o_ref):
    h = h_ref[...]
    m = jnp.max(h, axis=1, keepdims=True)
    e = jnp.exp(h - m)
    s = jnp.sum(e, axis=1, keepdims=True)
    o_ref[...] = h - m - jnp.log(s)


def _gcn_conv(x, W, b, src, dst, dis):
    N = x.shape[0]
    xs = dis[:, None] * x
    S = jax.ops.segment_sum(xs[src], dst, num_segments=N)
    agg = dis[:, None] * (S + xs)
    return agg @ W + b


def _batchnorm(x, g, b, eps=1e-5):
    m = jnp.mean(x, axis=0)
    v = jnp.var(x, axis=0)
    return (x - m) * jax.lax.rsqrt(v + eps) * g + b


def kernel(x, edge_index, W1, b1, g1, be1, W2, b2, g2, be2, W3, b3):
    N = x.shape[0]
    src = edge_index[0]
    dst = edge_index[1]
    ones = jnp.ones(src.shape[0], dtype=x.dtype)
    deg = jax.ops.segment_sum(ones, dst, num_segments=N) + 1.0
    dis = jax.lax.rsqrt(deg)
    h = jax.nn.relu(_batchnorm(_gcn_conv(x, W1, b1, src, dst, dis), g1, be1))
    h = jax.nn.relu(_batchnorm(_gcn_conv(h, W2, b2, src, dst, dis), g2, be2))
    h = _gcn_conv(h, W3, b3, src, dst, dis)
    out = pl.pallas_call(
        _ls_kernel,
        out_shape=jax.ShapeDtypeStruct((N, 3), jnp.float32),
        grid=(100,),
        in_specs=[pl.BlockSpec((N // 100, 3), lambda i: (i, 0))],
        out_specs=pl.BlockSpec((N // 100, 3), lambda i: (i, 0)),
    )(h)
    return out


# speedup vs baseline: 8.0240x; 4.9695x over previous
import functools

import jax
import jax.numpy as jnp
from jax import lax
from jax.experimental import pallas as pl
from jax.experimental.pallas import tpu as pltpu
from jax.experimental.pallas import tpu_sc as plsc

NC, NS = 2, 16
NW = NC * NS
BLK = 2000


def _sc_aggregate(hs, srcx, dstx, C=8000):
    NP, F = hs.shape
    NPF = NP * F
    EF = srcx.shape[0]
    per_w = EF // NS
    n_chunks = per_w // C
    ws = NPF // NS
    hs1 = hs.reshape(NPF)
    zeros1 = jnp.zeros((NPF,), jnp.float32)

    mesh = plsc.VectorSubcoreMesh(core_axis_name="c", subcore_axis_name="s",
                                  num_cores=1)

    @functools.partial(
        pl.kernel,
        out_type=jax.ShapeDtypeStruct((NPF,), jnp.float32),
        mesh=mesh,
        compiler_params=pltpu.CompilerParams(use_tc_tiling_on_sc=False),
        scratch_types=[
            pltpu.VMEM((C,), jnp.int32),
            pltpu.VMEM((C,), jnp.int32),
            pltpu.VMEM((C,), jnp.float32),
            pltpu.VMEM_SHARED((NPF,), jnp.float32),
            pltpu.VMEM_SHARED((NPF,), jnp.float32),
            pltpu.SemaphoreType.DMA,
        ],
    )
    def agg(hs_hbm, srcx_hbm, dstx_hbm, z_hbm, out_hbm, src_v, dst_v, vals_v,
            hs_sh, acc_sh, sem):
        sid = lax.axis_index("s")
        w0 = sid * ws
        pltpu.sync_copy(z_hbm.at[pl.ds(w0, ws)], acc_sh.at[pl.ds(w0, ws)])
        pltpu.sync_copy(hs_hbm.at[pl.ds(w0, ws)], hs_sh.at[pl.ds(w0, ws)])
        plsc.subcore_barrier()
        base = sid * per_w

        def body(i, carry):
            off = base + i * C
            pltpu.sync_copy(srcx_hbm.at[pl.ds(off, C)], src_v)
            pltpu.sync_copy(dstx_hbm.at[pl.ds(off, C)], dst_v)
            pltpu.async_copy(hs_sh.at[src_v], vals_v, sem).wait()
            pltpu.sync_copy(vals_v, acc_sh.at[dst_v], add=True)
            return carry

        lax.fori_loop(0, n_chunks, body, 0)
        plsc.subcore_barrier()
        pltpu.sync_copy(acc_sh.at[pl.ds(w0, ws)], out_hbm.at[pl.ds(w0, ws)])

    return agg(hs1, srcx, dstx, zeros1).reshape(NP, F)


def _expand_idx(idx, F):
    return (idx[:, None] * F
            + jnp.arange(F, dtype=jnp.int32)[None, :]).reshape(-1)


def _tc_prep(degP, x, NP):
    N, F = x.shape
    nb = N // BLK

    def body(degp_ref, x_ref, dis_ref, xs_ref):
        deg = degp_ref[...] + 1.0
        dis = lax.rsqrt(deg)
        dis_ref[...] = dis
        xs_ref[...] = dis * x_ref[...]

    return pl.pallas_call(
        body,
        grid=(nb,),
        in_specs=[
            pl.BlockSpec((BLK, 1), lambda i: (i, 0)),
            pl.BlockSpec((BLK, F), lambda i: (i, 0)),
        ],
        out_specs=[
            pl.BlockSpec((BLK, 1), lambda i: (i, 0)),
            pl.BlockSpec((BLK, F), lambda i: (i, 0)),
        ],
        out_shape=[
            jax.ShapeDtypeStruct((N, 1), jnp.float32),
            jax.ShapeDtypeStruct((NP, F), jnp.float32),
        ],
    )(degP, x)


def _tc_layer1(S1, xs, dis, W1, b1, g1, be1, W2, NP):
    N = dis.shape[0]
    nb = N // BLK
    Fm = W1.shape[1]
    Fo = W2.shape[1]

    def body(sp_ref, xs_ref, dis_ref, W1_ref, b1_ref, g1_ref, be1_ref,
             W2_ref, out_ref, s_ref):
        p = pl.program_id(0)
        i = pl.program_id(1)
        dis = dis_ref[...]
        agg = dis * (sp_ref[...] + xs_ref[...])
        pre = jnp.dot(agg, W1_ref[...],
                      preferred_element_type=jnp.float32) + b1_ref[...]

        @pl.when((p == 0) & (i == 0))
        def _():
            s_ref[...] = jnp.zeros_like(s_ref)

        @pl.when(p == 0)
        def _():
            s_ref[0:1, 0:Fm] += jnp.sum(pre, axis=0, keepdims=True)
            s_ref[1:2, 0:Fm] += jnp.sum(pre * pre, axis=0, keepdims=True)

        @pl.when(p == 1)
        def _():
            n = jnp.float32(N)
            mean = s_ref[0:1, 0:Fm] / n
            var = s_ref[1:2, 0:Fm] / n - mean * mean
            k = lax.rsqrt(var + 1e-5)
            h = jnp.maximum((pre - mean) * (k * g1_ref[...]) + be1_ref[...],
                            0.0)
            out_ref[...] = dis * jnp.dot(h, W2_ref[...],
                                         preferred_element_type=jnp.float32)

    return pl.pallas_call(
        body,
        grid=(2, nb),
        in_specs=[
            pl.BlockSpec((BLK, 4), lambda p, i: (i, 0)),
            pl.BlockSpec((BLK, 4), lambda p, i: (i, 0)),
            pl.BlockSpec((BLK, 1), lambda p, i: (i, 0)),
            pl.BlockSpec((4, Fm), lambda p, i: (0, 0)),
            pl.BlockSpec((1, Fm), lambda p, i: (0, 0)),
            pl.BlockSpec((1, Fm), lambda p, i: (0, 0)),
            pl.BlockSpec((1, Fm), lambda p, i: (0, 0)),
            pl.BlockSpec((Fm, Fo), lambda p, i: (0, 0)),
        ],
        out_specs=pl.BlockSpec((BLK, Fo), lambda p, i: (i, 0)),
        out_shape=jax.ShapeDtypeStruct((NP, Fo), jnp.float32),
        scratch_shapes=[pltpu.VMEM((8, 128), jnp.float32)],
    )(S1, xs, dis, W1, b1.reshape(1, -1), g1.reshape(1, -1),
      be1.reshape(1, -1), W2)


def _tc_layer2(S2, hs2, dis, b2, g2, be2, W3p, NP):
    Fm = hs2.shape[1]
    N = dis.shape[0]
    nb = N // BLK
    Fo = W3p.shape[1]

    def body(sp_ref, hs_ref, dis_ref, b2_ref, g2_ref, be2_ref, W3_ref,
             out_ref, s_ref):
        p = pl.program_id(0)
        i = pl.program_id(1)
        dis = dis_ref[...]
        pre = dis * (sp_ref[...] + hs_ref[...]) + b2_ref[...]

        @pl.when((p == 0) & (i == 0))
        def _():
            s_ref[...] = jnp.zeros_like(s_ref)

        @pl.when(p == 0)
        def _():
            s_ref[0:1, 0:Fm] += jnp.sum(pre, axis=0, keepdims=True)
            s_ref[1:2, 0:Fm] += jnp.sum(pre * pre, axis=0, keepdims=True)

        @pl.when(p == 1)
        def _():
            n = jnp.float32(N)
            mean = s_ref[0:1, 0:Fm] / n
            var = s_ref[1:2, 0:Fm] / n - mean * mean
            k = lax.rsqrt(var + 1e-5)
            h = jnp.maximum((pre - mean) * (k * g2_ref[...]) + be2_ref[...],
                            0.0)
            out_ref[...] = dis * jnp.dot(h, W3_ref[...],
                                         preferred_element_type=jnp.float32)

    return pl.pallas_call(
        body,
        grid=(2, nb),
        in_specs=[
            pl.BlockSpec((BLK, Fm), lambda p, i: (i, 0)),
            pl.BlockSpec((BLK, Fm), lambda p, i: (i, 0)),
            pl.BlockSpec((BLK, 1), lambda p, i: (i, 0)),
            pl.BlockSpec((1, Fm), lambda p, i: (0, 0)),
            pl.BlockSpec((1, Fm), lambda p, i: (0, 0)),
            pl.BlockSpec((1, Fm), lambda p, i: (0, 0)),
            pl.BlockSpec((Fm, Fo), lambda p, i: (0, 0)),
        ],
        out_specs=pl.BlockSpec((BLK, Fo), lambda p, i: (i, 0)),
        out_shape=jax.ShapeDtypeStruct((NP, Fo), jnp.float32),
        scratch_shapes=[pltpu.VMEM((8, 128), jnp.float32)],
    )(S2, hs2, dis, b2.reshape(1, -1), g2.reshape(1, -1),
      be2.reshape(1, -1), W3p)


def _tc_layer3(S3, hs3, dis, b3):
    N = dis.shape[0]
    nb = N // BLK

    def body(sp_ref, hs_ref, dis_ref, b3_ref, out_ref):
        pre = (dis_ref[...] * (sp_ref[:, 0:3] + hs_ref[:, 0:3])
               + b3_ref[...])
        m = jnp.max(pre, axis=1, keepdims=True)
        e = jnp.exp(pre - m)
        s = jnp.sum(e, axis=1, keepdims=True)
        out_ref[...] = pre - m - jnp.log(s)

    return pl.pallas_call(
        body,
        grid=(nb,),
        in_specs=[
            pl.BlockSpec((BLK, 4), lambda i: (i, 0)),
            pl.BlockSpec((BLK, 4), lambda i: (i, 0)),
            pl.BlockSpec((BLK, 1), lambda i: (i, 0)),
            pl.BlockSpec((1, 3), lambda i: (0, 0)),
        ],
        out_specs=pl.BlockSpec((BLK, 3), lambda i: (i, 0)),
        out_shape=jax.ShapeDtypeStruct((N, 3), jnp.float32),
    )(S3, hs3, dis, b3.reshape(1, -1))


def kernel(x, edge_index, W1, b1, g1, be1, W2, b2, g2, be2, W3, b3):
    N = x.shape[0]
    NP = -(-N // 512) * 512
    src = edge_index[0]
    dst = edge_index[1]

    srcx4 = _expand_idx(src, 4)
    dstx4 = _expand_idx(dst, 4)
    srcx8 = _expand_idx(src, 8)
    dstx8 = _expand_idx(dst, 8)

    onesNP = jnp.ones((NP, 1), jnp.float32)
    degS = _sc_aggregate(onesNP, dst, dst)
    dis, xs = _tc_prep(degS, x, NP)
    S1 = _sc_aggregate(xs, srcx4, dstx4)
    hs2 = _tc_layer1(S1, xs, dis, W1, b1, g1, be1, W2, NP)
    S2 = _sc_aggregate(hs2, srcx8, dstx8)
    W3p = jnp.pad(W3, ((0, 0), (0, 1)))
    hs3 = _tc_layer2(S2, hs2, dis, b2, g2, be2, W3p, NP)
    S3 = _sc_aggregate(hs3, srcx4, dstx4)
    return _tc_layer3(S3, hs3, dis, b3)
